# baseline (speedup 1.0000x reference)
import functools

import jax
import jax.numpy as jnp
import numpy as np
from jax import lax
from jax.experimental import pallas as pl
from jax.experimental.pallas import tpu as pltpu
from jax.experimental.pallas import tpu_sc as plsc

_PC_RANGE = np.array([0.0, -40.0, -3.0, 70.4, 40.0, 1.0], dtype=np.float32)
_VOXEL_SIZE = np.array([0.05, 0.05, 0.1], dtype=np.float32)
_GRID = np.round((_PC_RANGE[3:] - _PC_RANGE[:3]) / _VOXEL_SIZE).astype(np.int32)

_N = 2_000_000
_T = _N // 128
_NC, _NS = 2, 16
_NW = _NC * _NS
_PT = 496
_LAST_T = _T - (_NW - 1) * _PT
_CT = 62
_K = _PT // _CT
_CP = _CT * 128

_RX, _RY, _RZ = (float(v) for v in _PC_RANGE[:3])
_SX, _SY, _SZ = (float(np.float32(1.0) / v) for v in _VOXEL_SIZE)
_GX, _GY, _GZ = (float(v) for v in _GRID)


def _chunk_compute(in_ref, out_ref):
    def tile_body(i, _):
        p = i * 128
        b = i * 512
        for g in range(8):
            o = g * 16
            x = in_ref[pl.ds(p + o, 16)]
            y = in_ref[pl.ds(_CP + p + o, 16)]
            z = in_ref[pl.ds(2 * _CP + p + o, 16)]
            tx = (x - _RX) * _SX
            ty = (y - _RY) * _SY
            tz = (z - _RZ) * _SZ
            ok = ((tx >= 0.0) & (tx < _GX)
                  & (ty >= 0.0) & (ty < _GY)
                  & (tz >= 0.0) & (tz < _GZ))
            out_ref[pl.ds(b + o, 16)] = jnp.where(ok, tz.astype(jnp.int32), -1)
            out_ref[pl.ds(b + 128 + o, 16)] = jnp.where(ok, ty.astype(jnp.int32), -1)
            out_ref[pl.ds(b + 256 + o, 16)] = jnp.where(ok, tx.astype(jnp.int32), -1)
        return 0

    lax.fori_loop(0, _CT, tile_body, 0)


@functools.partial(
    pl.kernel,
    out_type=jax.ShapeDtypeStruct((_N * 4,), jnp.int32),
    mesh=plsc.VectorSubcoreMesh(core_axis_name="c", subcore_axis_name="s"),
    scratch_types=[
        pltpu.VMEM((3 * _CP,), jnp.float32),
        pltpu.VMEM((3 * _CP,), jnp.float32),
        pltpu.VMEM((_CT * 512,), jnp.int32),
        pltpu.VMEM((_CT * 512,), jnp.int32),
        pltpu.SemaphoreType.DMA,
        pltpu.SemaphoreType.DMA,
        pltpu.SemaphoreType.DMA,
        pltpu.SemaphoreType.DMA,
    ],
    compiler_params=pltpu.CompilerParams(needs_layout_passes=False),
)
def _voxelize(planes_hbm, out_hbm, in0, in1, out0, out1, is0, is1, os0, os1):
    wid = lax.axis_index("s") * _NC + lax.axis_index("c")
    base = wid * _PT
    count = jnp.where(wid == _NW - 1, _LAST_T, _PT)
    ins, outs, isems, osems = (in0, in1), (out0, out1), (is0, is1), (os0, os1)

    def start(k):
        t0 = base + jnp.minimum(k * _CT, count - _CT)
        dmas = tuple(
            pltpu.async_copy(
                planes_hbm.at[pl.ds(c * _N + t0 * 128, _CP)],
                ins[k % 2].at[pl.ds(c * _CP, _CP)],
                isems[k % 2],
            )
            for c in range(3)
        )
        return dmas, t0

    in_dma, t0s = {}, {}
    in_dma[0], t0s[0] = start(0)
    out_dma = {}
    for k in range(_K):
        if k + 1 < _K:
            in_dma[k + 1], t0s[k + 1] = start(k + 1)
        for d in in_dma[k]:
            d.wait()
        if k >= 2:
            out_dma[k - 2].wait()
        _chunk_compute(ins[k % 2], outs[k % 2])
        out_dma[k] = pltpu.async_copy(
            outs[k % 2], out_hbm.at[pl.ds(t0s[k] * 512, _CT * 512)], osems[k % 2],
        )
    out_dma[_K - 2].wait()
    out_dma[_K - 1].wait()


def kernel(points):
    assert points.shape == (_N, 4)
    planes = points.T.reshape(_N * 4)
    out = _voxelize(planes)
    return out.reshape(_T, 4, 128).transpose(0, 2, 1).reshape(_N, 4)[:, :3]


# speedup vs baseline: 3.0164x; 3.0164x over previous
import functools

import jax
import jax.numpy as jnp
import numpy as np
from jax import lax
from jax.experimental import pallas as pl
from jax.experimental.pallas import tpu as pltpu
from jax.experimental.pallas import tpu_sc as plsc

_PC_RANGE = np.array([0.0, -40.0, -3.0, 70.4, 40.0, 1.0], dtype=np.float32)
_VOXEL_SIZE = np.array([0.05, 0.05, 0.1], dtype=np.float32)
_GRID = np.round((_PC_RANGE[3:] - _PC_RANGE[:3]) / _VOXEL_SIZE).astype(np.int32)

_N = 2_000_000
_T = _N // 128
_NC, _NS = 2, 16
_NW = _NC * _NS
_PT = 496
_LAST_T = _T - (_NW - 1) * _PT
_CT = 62
_K = _PT // _CT

_RX, _RY, _RZ = (float(v) for v in _PC_RANGE[:3])
_SX, _SY, _SZ = (float(np.float32(1.0) / v) for v in _VOXEL_SIZE)
_GX, _GY, _GZ = (float(v) for v in _GRID)


def _chunk_compute(in_ref, out_ref):
    def tile_body(i, _):
        b = i * 512
        for g in range(8):
            o = b + g * 16
            x = in_ref[pl.ds(o, 16)]
            y = in_ref[pl.ds(o + 128, 16)]
            z = in_ref[pl.ds(o + 256, 16)]
            tx = (x - _RX) * _SX
            ty = (y - _RY) * _SY
            tz = (z - _RZ) * _SZ
            ok = ((tx >= 0.0) & (tx < _GX)
                  & (ty >= 0.0) & (ty < _GY)
                  & (tz >= 0.0) & (tz < _GZ))
            out_ref[pl.ds(o, 16)] = jnp.where(ok, tz.astype(jnp.int32), -1)
            out_ref[pl.ds(o + 128, 16)] = jnp.where(ok, ty.astype(jnp.int32), -1)
            out_ref[pl.ds(o + 256, 16)] = jnp.where(ok, tx.astype(jnp.int32), -1)
        return 0

    lax.fori_loop(0, _CT, tile_body, 0)


@functools.partial(
    pl.kernel,
    out_type=jax.ShapeDtypeStruct((_N * 4,), jnp.int32),
    mesh=plsc.VectorSubcoreMesh(core_axis_name="c", subcore_axis_name="s"),
    scratch_types=[
        pltpu.VMEM((_CT * 512,), jnp.float32),
        pltpu.VMEM((_CT * 512,), jnp.float32),
        pltpu.VMEM((_CT * 512,), jnp.int32),
        pltpu.VMEM((_CT * 512,), jnp.int32),
        pltpu.SemaphoreType.DMA,
        pltpu.SemaphoreType.DMA,
        pltpu.SemaphoreType.DMA,
        pltpu.SemaphoreType.DMA,
    ],
    compiler_params=pltpu.CompilerParams(needs_layout_passes=False),
)
def _voxelize(points_hbm, out_hbm, in0, in1, out0, out1, is0, is1, os0, os1):
    wid = lax.axis_index("s") * _NC + lax.axis_index("c")
    base = wid * _PT
    count = jnp.where(wid == _NW - 1, _LAST_T, _PT)
    ins, outs, isems, osems = (in0, in1), (out0, out1), (is0, is1), (os0, os1)

    def start(k):
        e0 = (base + jnp.minimum(k * _CT, count - _CT)) * 512
        return pltpu.async_copy(
            points_hbm.at[pl.ds(e0, _CT * 512)], ins[k % 2], isems[k % 2],
        ), e0

    in_dma, e0s = {}, {}
    in_dma[0], e0s[0] = start(0)
    out_dma = {}
    for k in range(_K):
        if k + 1 < _K:
            in_dma[k + 1], e0s[k + 1] = start(k + 1)
        in_dma[k].wait()
        if k >= 2:
            out_dma[k - 2].wait()
        _chunk_compute(ins[k % 2], outs[k % 2])
        out_dma[k] = pltpu.async_copy(
            outs[k % 2], out_hbm.at[pl.ds(e0s[k], _CT * 512)], osems[k % 2],
        )
    out_dma[_K - 2].wait()
    out_dma[_K - 1].wait()


def kernel(points):
    assert points.shape == (_N, 4)
    flat = points.reshape(_T, 128, 4).transpose(0, 2, 1).reshape(_N * 4)
    out = _voxelize(flat)
    return out.reshape(_T, 4, 128).transpose(0, 2, 1).reshape(_N, 4)[:, :3]


# speedup vs baseline: 14.2100x; 4.7109x over previous
import functools

import jax
import jax.numpy as jnp
import numpy as np
from jax import lax
from jax.experimental import pallas as pl
from jax.experimental.pallas import tpu as pltpu
from jax.experimental.pallas import tpu_sc as plsc

_PC_RANGE = np.array([0.0, -40.0, -3.0, 70.4, 40.0, 1.0], dtype=np.float32)
_VOXEL_SIZE = np.array([0.05, 0.05, 0.1], dtype=np.float32)
_GRID = np.round((_PC_RANGE[3:] - _PC_RANGE[:3]) / _VOXEL_SIZE).astype(np.int32)

_N = 2_000_000
_T = _N // 128
_NC, _NS = 2, 16
_NW = _NC * _NS
_PT = 496
_LAST_T = _T - (_NW - 1) * _PT
_CT = 62
_K = _PT // _CT

_RX, _RY, _RZ = (float(v) for v in _PC_RANGE[:3])
_SX, _SY, _SZ = (float(np.float32(1.0) / v) for v in _VOXEL_SIZE)
_GX, _GY, _GZ = (float(v) for v in _GRID)


def _chunk_compute(in_ref, out_ref):
    def tile_body(i, _):
        for g in range(8):
            o = g * 16
            x = in_ref[i, 0, pl.ds(o, 16)]
            y = in_ref[i, 1, pl.ds(o, 16)]
            z = in_ref[i, 2, pl.ds(o, 16)]
            tx = (x - _RX) * _SX
            ty = (y - _RY) * _SY
            tz = (z - _RZ) * _SZ
            ok = ((tx >= 0.0) & (tx < _GX)
                  & (ty >= 0.0) & (ty < _GY)
                  & (tz >= 0.0) & (tz < _GZ))
            out_ref[i, 0, pl.ds(o, 16)] = jnp.where(ok, tz.astype(jnp.int32), -1)
            out_ref[i, 1, pl.ds(o, 16)] = jnp.where(ok, ty.astype(jnp.int32), -1)
            out_ref[i, 2, pl.ds(o, 16)] = jnp.where(ok, tx.astype(jnp.int32), -1)
        return 0

    lax.fori_loop(0, _CT, tile_body, 0)


@functools.partial(
    pl.kernel,
    out_type=jax.ShapeDtypeStruct((_T, 4, 128), jnp.int32),
    mesh=plsc.VectorSubcoreMesh(core_axis_name="c", subcore_axis_name="s"),
    scratch_types=[
        pltpu.VMEM((_CT, 4, 128), jnp.float32),
        pltpu.VMEM((_CT, 4, 128), jnp.float32),
        pltpu.VMEM((_CT, 4, 128), jnp.int32),
        pltpu.VMEM((_CT, 4, 128), jnp.int32),
        pltpu.SemaphoreType.DMA,
        pltpu.SemaphoreType.DMA,
        pltpu.SemaphoreType.DMA,
        pltpu.SemaphoreType.DMA,
    ],
    compiler_params=pltpu.CompilerParams(needs_layout_passes=False),
)
def _voxelize(points_hbm, out_hbm, in0, in1, out0, out1, is0, is1, os0, os1):
    wid = lax.axis_index("s") * _NC + lax.axis_index("c")
    base = wid * _PT
    count = jnp.where(wid == _NW - 1, _LAST_T, _PT)
    ins, outs, isems, osems = (in0, in1), (out0, out1), (is0, is1), (os0, os1)

    def start(k):
        t0 = base + jnp.minimum(k * _CT, count - _CT)
        return pltpu.async_copy(
            points_hbm.at[pl.ds(t0, _CT)], ins[k % 2], isems[k % 2],
        ), t0

    in_dma, t0s = {}, {}
    in_dma[0], t0s[0] = start(0)
    out_dma = {}
    for k in range(_K):
        if k + 1 < _K:
            in_dma[k + 1], t0s[k + 1] = start(k + 1)
        in_dma[k].wait()
        if k >= 2:
            out_dma[k - 2].wait()
        _chunk_compute(ins[k % 2], outs[k % 2])
        out_dma[k] = pltpu.async_copy(
            outs[k % 2], out_hbm.at[pl.ds(t0s[k], _CT)], osems[k % 2],
        )
    out_dma[_K - 2].wait()
    out_dma[_K - 1].wait()


def kernel(points):
    assert points.shape == (_N, 4)
    tiles = points.reshape(_T, 128, 4).transpose(0, 2, 1)
    out = _voxelize(tiles)
    return out.transpose(0, 2, 1).reshape(_N, 4)[:, :3]


# speedup vs baseline: 14.9407x; 1.0514x over previous
import functools

import jax
import jax.numpy as jnp
import numpy as np
from jax import lax
from jax.experimental import pallas as pl
from jax.experimental.pallas import tpu as pltpu
from jax.experimental.pallas import tpu_sc as plsc

_PC_RANGE = np.array([0.0, -40.0, -3.0, 70.4, 40.0, 1.0], dtype=np.float32)
_VOXEL_SIZE = np.array([0.05, 0.05, 0.1], dtype=np.float32)
_GRID = np.round((_PC_RANGE[3:] - _PC_RANGE[:3]) / _VOXEL_SIZE).astype(np.int32)

_N = 2_000_000
_T = _N // 128
_NC, _NS = 2, 16
_NW = _NC * _NS
_PT = 496
_LAST_T = _T - (_NW - 1) * _PT
_CT = 62
_K = _PT // _CT

_RX, _RY, _RZ = (float(v) for v in _PC_RANGE[:3])
_SX, _SY, _SZ = (float(np.float32(1.0) / v) for v in _VOXEL_SIZE)
_BX, _BY, _BZ = (int(np.float32(v).view(np.uint32)) for v in _GRID)


def _chunk_compute(in_ref, out_ref):
    def tile_body(i, _):
        for g in range(8):
            o = g * 16
            x = in_ref[i, 0, pl.ds(o, 16)]
            y = in_ref[i, 1, pl.ds(o, 16)]
            z = in_ref[i, 2, pl.ds(o, 16)]
            tx = (x - _RX) * _SX
            ty = (y - _RY) * _SY
            tz = (z - _RZ) * _SZ
            ok = ((lax.bitcast_convert_type(tx, jnp.uint32) < _BX)
                  & (lax.bitcast_convert_type(ty, jnp.uint32) < _BY)
                  & (lax.bitcast_convert_type(tz, jnp.uint32) < _BZ))
            out_ref[i, 0, pl.ds(o, 16)] = jnp.where(ok, tz.astype(jnp.int32), -1)
            out_ref[i, 1, pl.ds(o, 16)] = jnp.where(ok, ty.astype(jnp.int32), -1)
            out_ref[i, 2, pl.ds(o, 16)] = jnp.where(ok, tx.astype(jnp.int32), -1)
        return 0

    lax.fori_loop(0, _CT, tile_body, 0)


@functools.partial(
    pl.kernel,
    out_type=jax.ShapeDtypeStruct((_T, 4, 128), jnp.int32),
    mesh=plsc.VectorSubcoreMesh(core_axis_name="c", subcore_axis_name="s"),
    scratch_types=[
        pltpu.VMEM((_CT, 4, 128), jnp.float32),
        pltpu.VMEM((_CT, 4, 128), jnp.float32),
        pltpu.VMEM((_CT, 4, 128), jnp.int32),
        pltpu.VMEM((_CT, 4, 128), jnp.int32),
        pltpu.SemaphoreType.DMA,
        pltpu.SemaphoreType.DMA,
        pltpu.SemaphoreType.DMA,
        pltpu.SemaphoreType.DMA,
    ],
    compiler_params=pltpu.CompilerParams(needs_layout_passes=False),
)
def _voxelize(points_hbm, out_hbm, in0, in1, out0, out1, is0, is1, os0, os1):
    wid = lax.axis_index("s") * _NC + lax.axis_index("c")
    base = wid * _PT
    count = jnp.where(wid == _NW - 1, _LAST_T, _PT)
    ins, outs, isems, osems = (in0, in1), (out0, out1), (is0, is1), (os0, os1)

    def start(k):
        t0 = base + jnp.minimum(k * _CT, count - _CT)
        return pltpu.async_copy(
            points_hbm.at[pl.ds(t0, _CT)], ins[k % 2], isems[k % 2],
        ), t0

    in_dma, t0s = {}, {}
    in_dma[0], t0s[0] = start(0)
    out_dma = {}
    for k in range(_K):
        if k + 1 < _K:
            in_dma[k + 1], t0s[k + 1] = start(k + 1)
        in_dma[k].wait()
        if k >= 2:
            out_dma[k - 2].wait()
        _chunk_compute(ins[k % 2], outs[k % 2])
        out_dma[k] = pltpu.async_copy(
            outs[k % 2], out_hbm.at[pl.ds(t0s[k], _CT)], osems[k % 2],
        )
    out_dma[_K - 2].wait()
    out_dma[_K - 1].wait()


def kernel(points):
    assert points.shape == (_N, 4)
    tiles = points.reshape(_T, 128, 4).transpose(0, 2, 1)
    out = _voxelize(tiles)
    return out.transpose(0, 2, 1).reshape(_N, 4)[:, :3]
